# baseline (speedup 1.0000x reference)
import jax
import jax.numpy as jnp
from jax.experimental import pallas as pl

_NB = 2
_N = 10000
_FT = 128
_HID = 64
_TILE = 200


def _project_kernel(f_ref, s_ref, w_ref, z_ref):
    w = w_ref[0]
    z_ref[0, :, 0:_HID] = jnp.dot(f_ref[0], w, preferred_element_type=jnp.float32)
    z_ref[0, :, _HID:2 * _HID] = jnp.dot(s_ref[0], w, preferred_element_type=jnp.float32)


def _propagate_kernel(a_ref, z_ref, h_ref):
    y = jnp.dot(a_ref[0], z_ref[0], preferred_element_type=jnp.float32,
                precision=jax.lax.Precision.DEFAULT)
    h_ref[0] = jnp.maximum(y, 0.0)


def _epilogue_kernel(h_ref, wb_ref, hmat_ref, sc_ref, reg_ref):
    wb = wb_ref[...]
    hm = hmat_ref[0]
    for g in range(_NB):
        h1 = h_ref[g, :, 0:_HID]
        h2 = h_ref[g, :, _HID:2 * _HID]
        cm = jax.nn.sigmoid(jnp.mean(h1, axis=0, keepdims=True))
        v = jax.lax.dot_general(cm, wb, (((1,), (1,)), ((), ())),
                                preferred_element_type=jnp.float32)
        sc1 = jnp.sum(h1 * v, axis=1, keepdims=True)
        sc2 = jnp.sum(h2 * v, axis=1, keepdims=True)
        sc_ref[g] = jnp.concatenate([sc1, sc2], axis=1)
    h1a = 0.5 * (h_ref[0, :, 0:_HID] + h_ref[1, :, 0:_HID])
    h2a = 0.5 * (h_ref[0, :, _HID:2 * _HID] + h_ref[1, :, _HID:2 * _HID])
    pos = jnp.sum((hm - h1a) ** 2)
    neg = jnp.sum((hm - h2a) ** 2)
    reg_ref[:, :] = jnp.reshape(pos - neg, (1, 1))


def kernel(feature, adj, shuf, sparse, msk, samp_bias1, samp_bias2, W_gcn, W_bil, b_bil, H):
    f = feature.reshape(_NB, _N, _FT)
    s = shuf.reshape(_NB, _N, _FT)
    a = adj.reshape(_NB, _N, _N)

    z = pl.pallas_call(
        _project_kernel,
        grid=(_NB,),
        in_specs=[
            pl.BlockSpec((1, _N, _FT), lambda g: (g, 0, 0)),
            pl.BlockSpec((1, _N, _FT), lambda g: (g, 0, 0)),
            pl.BlockSpec((1, _FT, _HID), lambda g: (g, 0, 0)),
        ],
        out_specs=pl.BlockSpec((1, _N, 2 * _HID), lambda g: (g, 0, 0)),
        out_shape=jax.ShapeDtypeStruct((_NB, _N, 2 * _HID), jnp.float32),
    )(f, s, W_gcn)

    h = pl.pallas_call(
        _propagate_kernel,
        grid=(_NB, _N // _TILE),
        in_specs=[
            pl.BlockSpec((1, _TILE, _N), lambda g, t: (g, t, 0)),
            pl.BlockSpec((1, _N, 2 * _HID), lambda g, t: (g, 0, 0)),
        ],
        out_specs=pl.BlockSpec((1, _TILE, 2 * _HID), lambda g, t: (g, t, 0)),
        out_shape=jax.ShapeDtypeStruct((_NB, _N, 2 * _HID), jnp.float32),
    )(a, z)

    sc, reg = pl.pallas_call(
        _epilogue_kernel,
        in_specs=[
            pl.BlockSpec((_NB, _N, 2 * _HID), lambda: (0, 0, 0)),
            pl.BlockSpec((_HID, _HID), lambda: (0, 0)),
            pl.BlockSpec((1, _N, _HID), lambda: (0, 0, 0)),
        ],
        out_specs=[
            pl.BlockSpec((_NB, _N, 2), lambda: (0, 0, 0)),
            pl.BlockSpec((1, 1), lambda: (0, 0)),
        ],
        out_shape=[
            jax.ShapeDtypeStruct((_NB, _N, 2), jnp.float32),
            jax.ShapeDtypeStruct((1, 1), jnp.float32),
        ],
    )(h, W_bil, H)

    logits = jnp.transpose(sc, (0, 2, 1)).reshape(_NB, 1, 2 * _N)
    logits = logits + jnp.concatenate([samp_bias1, samp_bias2], axis=1)[None] + b_bil
    reg_loss = reg[0, 0]
    return (logits, reg_loss)


# speedup vs baseline: 1.0327x; 1.0327x over previous
import jax
import jax.numpy as jnp
from jax.experimental import pallas as pl
from jax.experimental.pallas import tpu as pltpu

_NB = 2
_N = 10000
_FT = 128
_HID = 64
_TILE = 200


def _propagate_kernel(f_ref, s_ref, w_ref, a_ref, h_ref, z_scr):
    t = pl.program_id(1)

    @pl.when(t == 0)
    def _():
        w = w_ref[0]
        z_scr[:, 0:_HID] = jnp.dot(f_ref[0], w, preferred_element_type=jnp.float32)
        z_scr[:, _HID:2 * _HID] = jnp.dot(s_ref[0], w, preferred_element_type=jnp.float32)

    y = jnp.dot(a_ref[0], z_scr[...], preferred_element_type=jnp.float32)
    h_ref[0] = jnp.maximum(y, 0.0)


def _epilogue_kernel(h_ref, wb_ref, hmat_ref, sc_ref, reg_ref):
    wb = wb_ref[...]
    hm = hmat_ref[0]
    for g in range(_NB):
        h1 = h_ref[g, :, 0:_HID]
        h2 = h_ref[g, :, _HID:2 * _HID]
        cm = jax.nn.sigmoid(jnp.mean(h1, axis=0, keepdims=True))
        v = jax.lax.dot_general(cm, wb, (((1,), (1,)), ((), ())),
                                preferred_element_type=jnp.float32)
        sc1 = jax.lax.dot_general(h1, v, (((1,), (1,)), ((), ())),
                                  preferred_element_type=jnp.float32)
        sc2 = jax.lax.dot_general(h2, v, (((1,), (1,)), ((), ())),
                                  preferred_element_type=jnp.float32)
        sc_ref[g] = jnp.concatenate([sc1, sc2], axis=1)
    h1a = 0.5 * (h_ref[0, :, 0:_HID] + h_ref[1, :, 0:_HID])
    h2a = 0.5 * (h_ref[0, :, _HID:2 * _HID] + h_ref[1, :, _HID:2 * _HID])
    pos = jnp.sum((hm - h1a) ** 2)
    neg = jnp.sum((hm - h2a) ** 2)
    reg_ref[:, :] = jnp.reshape(pos - neg, (1, 1))


def kernel(feature, adj, shuf, sparse, msk, samp_bias1, samp_bias2, W_gcn, W_bil, b_bil, H):
    f = feature.reshape(_NB, _N, _FT)
    s = shuf.reshape(_NB, _N, _FT)
    a = adj.reshape(_NB, _N, _N)

    h = pl.pallas_call(
        _propagate_kernel,
        grid=(_NB, _N // _TILE),
        in_specs=[
            pl.BlockSpec((1, _N, _FT), lambda g, t: (g, 0, 0)),
            pl.BlockSpec((1, _N, _FT), lambda g, t: (g, 0, 0)),
            pl.BlockSpec((1, _FT, _HID), lambda g, t: (g, 0, 0)),
            pl.BlockSpec((1, _TILE, _N), lambda g, t: (g, t, 0)),
        ],
        out_specs=pl.BlockSpec((1, _TILE, 2 * _HID), lambda g, t: (g, t, 0)),
        out_shape=jax.ShapeDtypeStruct((_NB, _N, 2 * _HID), jnp.float32),
        scratch_shapes=[pltpu.VMEM((_N, 2 * _HID), jnp.float32)],
    )(f, s, W_gcn, a)

    sc, reg = pl.pallas_call(
        _epilogue_kernel,
        in_specs=[
            pl.BlockSpec((_NB, _N, 2 * _HID), lambda: (0, 0, 0)),
            pl.BlockSpec((_HID, _HID), lambda: (0, 0)),
            pl.BlockSpec((1, _N, _HID), lambda: (0, 0, 0)),
        ],
        out_specs=[
            pl.BlockSpec((_NB, _N, 2), lambda: (0, 0, 0)),
            pl.BlockSpec((1, 1), lambda: (0, 0)),
        ],
        out_shape=[
            jax.ShapeDtypeStruct((_NB, _N, 2), jnp.float32),
            jax.ShapeDtypeStruct((1, 1), jnp.float32),
        ],
    )(h, W_bil, H)

    logits = jnp.transpose(sc, (0, 2, 1)).reshape(_NB, 1, 2 * _N)
    logits = logits + jnp.concatenate([samp_bias1, samp_bias2], axis=1)[None] + b_bil
    reg_loss = reg[0, 0]
    return (logits, reg_loss)


# speedup vs baseline: 1.0423x; 1.0093x over previous
import jax
import jax.numpy as jnp
from jax.experimental import pallas as pl
from jax.experimental.pallas import tpu as pltpu

_NB = 2
_N = 10000
_FT = 128
_HID = 64
_TILE = 400


def _propagate_kernel(f_ref, s_ref, w_ref, a_ref, h_ref, z_scr):
    t = pl.program_id(1)

    @pl.when(t == 0)
    def _():
        w = w_ref[0]
        z_scr[:, 0:_HID] = jnp.dot(f_ref[0], w, preferred_element_type=jnp.float32)
        z_scr[:, _HID:2 * _HID] = jnp.dot(s_ref[0], w, preferred_element_type=jnp.float32)

    y = jnp.dot(a_ref[0], z_scr[...], preferred_element_type=jnp.float32)
    h_ref[0] = jnp.maximum(y, 0.0)


def _epilogue_kernel(h_ref, wb_ref, hmat_ref, sc_ref, reg_ref):
    wb = wb_ref[...]
    hm = hmat_ref[0]
    for g in range(_NB):
        h1 = h_ref[g, :, 0:_HID]
        h2 = h_ref[g, :, _HID:2 * _HID]
        cm = jax.nn.sigmoid(jnp.mean(h1, axis=0, keepdims=True))
        v = jax.lax.dot_general(cm, wb, (((1,), (1,)), ((), ())),
                                preferred_element_type=jnp.float32)
        sc1 = jax.lax.dot_general(h1, v, (((1,), (1,)), ((), ())),
                                  preferred_element_type=jnp.float32)
        sc2 = jax.lax.dot_general(h2, v, (((1,), (1,)), ((), ())),
                                  preferred_element_type=jnp.float32)
        sc_ref[g] = jnp.concatenate([sc1, sc2], axis=1)
    h1a = 0.5 * (h_ref[0, :, 0:_HID] + h_ref[1, :, 0:_HID])
    h2a = 0.5 * (h_ref[0, :, _HID:2 * _HID] + h_ref[1, :, _HID:2 * _HID])
    pos = jnp.sum((hm - h1a) ** 2)
    neg = jnp.sum((hm - h2a) ** 2)
    reg_ref[:, :] = jnp.reshape(pos - neg, (1, 1))


def kernel(feature, adj, shuf, sparse, msk, samp_bias1, samp_bias2, W_gcn, W_bil, b_bil, H):
    f = feature.reshape(_NB, _N, _FT)
    s = shuf.reshape(_NB, _N, _FT)
    a = adj.reshape(_NB, _N, _N)

    h = pl.pallas_call(
        _propagate_kernel,
        grid=(_NB, _N // _TILE),
        in_specs=[
            pl.BlockSpec((1, _N, _FT), lambda g, t: (g, 0, 0)),
            pl.BlockSpec((1, _N, _FT), lambda g, t: (g, 0, 0)),
            pl.BlockSpec((1, _FT, _HID), lambda g, t: (g, 0, 0)),
            pl.BlockSpec((1, _TILE, _N), lambda g, t: (g, t, 0)),
        ],
        out_specs=pl.BlockSpec((1, _TILE, 2 * _HID), lambda g, t: (g, t, 0)),
        out_shape=jax.ShapeDtypeStruct((_NB, _N, 2 * _HID), jnp.float32),
        scratch_shapes=[pltpu.VMEM((_N, 2 * _HID), jnp.float32)],
    )(f, s, W_gcn, a)

    sc, reg = pl.pallas_call(
        _epilogue_kernel,
        in_specs=[
            pl.BlockSpec((_NB, _N, 2 * _HID), lambda: (0, 0, 0)),
            pl.BlockSpec((_HID, _HID), lambda: (0, 0)),
            pl.BlockSpec((1, _N, _HID), lambda: (0, 0, 0)),
        ],
        out_specs=[
            pl.BlockSpec((_NB, _N, 2), lambda: (0, 0, 0)),
            pl.BlockSpec((1, 1), lambda: (0, 0)),
        ],
        out_shape=[
            jax.ShapeDtypeStruct((_NB, _N, 2), jnp.float32),
            jax.ShapeDtypeStruct((1, 1), jnp.float32),
        ],
    )(h, W_bil, H)

    logits = jnp.transpose(sc, (0, 2, 1)).reshape(_NB, 1, 2 * _N)
    logits = logits + jnp.concatenate([samp_bias1, samp_bias2], axis=1)[None] + b_bil
    reg_loss = reg[0, 0]
    return (logits, reg_loss)
